# Initial kernel scaffold; baseline (speedup 1.0000x reference)
import functools

import jax
import jax.numpy as jnp
from jax import lax
from jax.experimental import pallas as pl
from jax.experimental.pallas import tpu as pltpu
from jax.experimental.pallas import tpu_sc as plsc

N = 10000
E = 320000
D_IN = 128
D_HID = 64
D_OUT = 1

NC = 2
NS = 16
NW = NC * NS
K = 128
CHUNKS = 80
E_PAD = NW * CHUNKS * K
N_ACC = 10240


def _make_edge_agg(feat):
    mesh = plsc.VectorSubcoreMesh(core_axis_name="c", subcore_axis_name="s")
    rows_per_tile = N_ACC // NS

    @functools.partial(
        pl.kernel,
        out_type=jax.ShapeDtypeStruct((NC, N_ACC, feat), jnp.float32),
        mesh=mesh,
        scratch_types=[
            pltpu.VMEM((CHUNKS, K), jnp.int32),
            pltpu.VMEM((CHUNKS, K), jnp.int32),
            pltpu.VMEM((K, feat), jnp.float32),
            pltpu.VMEM_SHARED((N_ACC, feat), jnp.float32),
        ],
    )
    def agg(table_hbm, srcs_hbm, dsts_hbm, zeros_hbm, out_hbm,
            src_v, dst_v, rows_v, acc_sh):
        c = lax.axis_index("c")
        s = lax.axis_index("s")
        wid = c * NS + s
        pltpu.sync_copy(zeros_hbm.at[pl.ds(s * rows_per_tile, rows_per_tile)],
                        acc_sh.at[pl.ds(s * rows_per_tile, rows_per_tile)])
        pltpu.sync_copy(srcs_hbm.at[wid], src_v)
        pltpu.sync_copy(dsts_hbm.at[wid], dst_v)
        plsc.subcore_barrier()

        def body(j, carry):
            pltpu.sync_copy(table_hbm.at[src_v.at[j]], rows_v)
            pltpu.sync_copy(rows_v, acc_sh.at[dst_v.at[j]], add=True)
            return carry

        lax.fori_loop(0, CHUNKS, body, 0)
        plsc.subcore_barrier()
        pltpu.sync_copy(acc_sh.at[pl.ds(s * rows_per_tile, rows_per_tile)],
                        out_hbm.at[c].at[pl.ds(s * rows_per_tile, rows_per_tile)])

    return agg


_agg64 = _make_edge_agg(D_HID)
_agg1 = _make_edge_agg(1)


def _mm1_body(x_ref, w1_ref, degp_ref, t1_ref, dinv_ref):
    deg = degp_ref[0, :N, :] + degp_ref[1, :N, :] + 1.0
    dinv = lax.rsqrt(deg)
    mm = jnp.dot(x_ref[...], w1_ref[...], preferred_element_type=jnp.float32)
    t1_ref[...] = mm * dinv
    dinv_ref[...] = dinv


def _mm2_body(aggp_ref, t1_ref, dinv_ref, w2_ref, b1_ref, t2_ref):
    aggsum = aggp_ref[0, :N, :] + aggp_ref[1, :N, :] + t1_ref[...]
    h = jnp.maximum(aggsum * dinv_ref[...] + b1_ref[...], 0.0)
    mm = jnp.dot(h, w2_ref[...], preferred_element_type=jnp.float32)
    t2_ref[...] = mm * dinv_ref[...]


def _fin_body(aggp_ref, t2_ref, dinv_ref, b2_ref, out_ref):
    a = aggp_ref[0, :N, :] + aggp_ref[1, :N, :] + t2_ref[...]
    out_ref[...] = jax.nn.sigmoid(a * dinv_ref[...] + b2_ref[...])


def kernel(x, edge_index, W1, b1, W2, b2):
    src = edge_index[0].astype(jnp.int32)
    dst = edge_index[1].astype(jnp.int32)

    pad = E_PAD - E
    pr = jnp.arange(pad, dtype=jnp.int32)
    pad_src = (pr * 997) % N
    pad_dst = N + pr % (N_ACC - N)
    srcs = jnp.concatenate([src, pad_src]).reshape(NW, CHUNKS, K)
    dsts = jnp.concatenate([dst, pad_dst]).reshape(NW, CHUNKS, K)

    zeros64 = jnp.zeros((N_ACC, D_HID), jnp.float32)
    zeros1 = jnp.zeros((N_ACC, 1), jnp.float32)
    ones_tbl = jnp.ones((N, 1), jnp.float32)

    degp = _agg1(ones_tbl, srcs, dsts, zeros1)

    t1, dinv = pl.pallas_call(
        _mm1_body,
        out_shape=(
            jax.ShapeDtypeStruct((N, D_HID), jnp.float32),
            jax.ShapeDtypeStruct((N, 1), jnp.float32),
        ),
    )(x, W1, degp)

    agg1p = _agg64(t1, srcs, dsts, zeros64)

    t2 = pl.pallas_call(
        _mm2_body,
        out_shape=jax.ShapeDtypeStruct((N, 1), jnp.float32),
    )(agg1p, t1, dinv, W2, b1)

    agg2p = _agg1(t2, srcs, dsts, zeros1)

    out = pl.pallas_call(
        _fin_body,
        out_shape=jax.ShapeDtypeStruct((N, D_OUT), jnp.float32),
    )(agg2p, t2, dinv, b2)
    return out


# speedup vs baseline: 26.9056x; 26.9056x over previous
import functools

import jax
import jax.numpy as jnp
from jax import lax
from jax.experimental import pallas as pl
from jax.experimental.pallas import tpu as pltpu
from jax.experimental.pallas import tpu_sc as plsc

N = 10000
E = 320000
D_IN = 128
D_HID = 64
D_OUT = 1

NC = 2
NS = 16
NW = NC * NS
K = 128
CHUNKS = 80
E_PAD = NW * CHUNKS * K
N_ACC = 10240


def _make_edge_agg(feat):
    mesh = plsc.VectorSubcoreMesh(core_axis_name="c", subcore_axis_name="s")
    rows_per_tile = N_ACC // NS

    @functools.partial(
        pl.kernel,
        out_type=jax.ShapeDtypeStruct((NC, N_ACC, feat), jnp.float32),
        mesh=mesh,
        scratch_types=[
            pltpu.VMEM((CHUNKS, K), jnp.int32),
            pltpu.VMEM((CHUNKS, K), jnp.int32),
            pltpu.VMEM((K, feat), jnp.float32),
            pltpu.VMEM_SHARED((N_ACC, feat), jnp.float32),
        ],
        compiler_params=pltpu.CompilerParams(use_tc_tiling_on_sc=False),
    )
    def agg(table_hbm, srcs_hbm, dsts_hbm, zeros_hbm, out_hbm,
            src_v, dst_v, rows_v, acc_sh):
        c = lax.axis_index("c")
        s = lax.axis_index("s")
        wid = c * NS + s
        pltpu.sync_copy(zeros_hbm.at[pl.ds(s * rows_per_tile, rows_per_tile)],
                        acc_sh.at[pl.ds(s * rows_per_tile, rows_per_tile)])
        pltpu.sync_copy(srcs_hbm.at[wid], src_v)
        pltpu.sync_copy(dsts_hbm.at[wid], dst_v)
        plsc.subcore_barrier()

        def body(j, carry):
            pltpu.sync_copy(table_hbm.at[src_v.at[j]], rows_v)
            pltpu.sync_copy(rows_v, acc_sh.at[dst_v.at[j]], add=True)
            return carry

        lax.fori_loop(0, CHUNKS, body, 0)
        plsc.subcore_barrier()
        pltpu.sync_copy(acc_sh.at[pl.ds(s * rows_per_tile, rows_per_tile)],
                        out_hbm.at[c].at[pl.ds(s * rows_per_tile, rows_per_tile)])

    return agg


_agg64 = _make_edge_agg(D_HID)
_agg1 = _make_edge_agg(1)


def _mm1_body(x_ref, w1_ref, degp_ref, t1_ref, dinv_ref):
    deg = degp_ref[0, :N, :] + degp_ref[1, :N, :] + 1.0
    dinv = lax.rsqrt(deg)
    mm = jnp.dot(x_ref[...], w1_ref[...], preferred_element_type=jnp.float32)
    t1_ref[...] = mm * dinv
    dinv_ref[...] = dinv


def _mm2_body(aggp_ref, t1_ref, dinv_ref, w2_ref, b1_ref, t2_ref):
    aggsum = aggp_ref[0, :N, :] + aggp_ref[1, :N, :] + t1_ref[...]
    h = jnp.maximum(aggsum * dinv_ref[...] + b1_ref[...], 0.0)
    mm = jnp.dot(h, w2_ref[...], preferred_element_type=jnp.float32)
    t2_ref[...] = mm * dinv_ref[...]


def _fin_body(aggp_ref, t2_ref, dinv_ref, b2_ref, out_ref):
    a = aggp_ref[0, :N, :] + aggp_ref[1, :N, :] + t2_ref[...]
    out_ref[...] = jax.nn.sigmoid(a * dinv_ref[...] + b2_ref[...])


def kernel(x, edge_index, W1, b1, W2, b2):
    src = edge_index[0].astype(jnp.int32)
    dst = edge_index[1].astype(jnp.int32)

    pad = E_PAD - E
    pr = jnp.arange(pad, dtype=jnp.int32)
    pad_src = (pr * 997) % N
    pad_dst = N + pr % (N_ACC - N)
    srcs = jnp.concatenate([src, pad_src]).reshape(NW, CHUNKS, K)
    dsts = jnp.concatenate([dst, pad_dst]).reshape(NW, CHUNKS, K)

    zeros64 = jnp.zeros((N_ACC, D_HID), jnp.float32)
    zeros1 = jnp.zeros((N_ACC, 1), jnp.float32)
    ones_tbl = jnp.ones((N, 1), jnp.float32)

    degp = _agg1(ones_tbl, srcs, dsts, zeros1)

    t1, dinv = pl.pallas_call(
        _mm1_body,
        out_shape=(
            jax.ShapeDtypeStruct((N, D_HID), jnp.float32),
            jax.ShapeDtypeStruct((N, 1), jnp.float32),
        ),
    )(x, W1, degp)

    agg1p = _agg64(t1, srcs, dsts, zeros64)

    t2 = pl.pallas_call(
        _mm2_body,
        out_shape=jax.ShapeDtypeStruct((N, 1), jnp.float32),
    )(agg1p, t1, dinv, W2, b1)

    agg2p = _agg1(t2, srcs, dsts, zeros1)

    out = pl.pallas_call(
        _fin_body,
        out_shape=jax.ShapeDtypeStruct((N, D_OUT), jnp.float32),
    )(agg2p, t2, dinv, b2)
    return out


# speedup vs baseline: 37.6930x; 1.4009x over previous
import functools

import jax
import jax.numpy as jnp
from jax import lax
from jax.experimental import pallas as pl
from jax.experimental.pallas import tpu as pltpu
from jax.experimental.pallas import tpu_sc as plsc

N = 10000
E = 320000
D_IN = 128
D_HID = 64
D_OUT = 1

NC = 2
NS = 16
NW = NC * NS
K = 128
CHUNKS = 80
E_PAD = NW * CHUNKS * K
N_ACC = 10240
ROWS_PER_TILE = N_ACC // NS

_MESH = plsc.VectorSubcoreMesh(core_axis_name="c", subcore_axis_name="s")
_SC_PARAMS = pltpu.CompilerParams(use_tc_tiling_on_sc=False)


def _acc_slice(s):
    return pl.ds(s * ROWS_PER_TILE, ROWS_PER_TILE)


def _make_edge_agg(feat):
    @functools.partial(
        pl.kernel,
        out_type=jax.ShapeDtypeStruct((NC, N_ACC, feat), jnp.float32),
        mesh=_MESH,
        scratch_types=[
            pltpu.VMEM((CHUNKS, K), jnp.int32),
            pltpu.VMEM((CHUNKS, K), jnp.int32),
            pltpu.VMEM((K, feat), jnp.float32),
            pltpu.VMEM_SHARED((N_ACC, feat), jnp.float32),
            pltpu.VMEM_SHARED((N_ACC, feat), jnp.float32),
        ],
        compiler_params=_SC_PARAMS,
    )
    def agg(table_hbm, srcs_hbm, dsts_hbm, zeros_hbm, out_hbm,
            src_v, dst_v, rows_v, acc_sh, tbl_sh):
        c = lax.axis_index("c")
        s = lax.axis_index("s")
        wid = c * NS + s
        pltpu.sync_copy(zeros_hbm.at[_acc_slice(s)], acc_sh.at[_acc_slice(s)])
        pltpu.sync_copy(table_hbm.at[_acc_slice(s)], tbl_sh.at[_acc_slice(s)])
        pltpu.sync_copy(srcs_hbm.at[wid], src_v)
        pltpu.sync_copy(dsts_hbm.at[wid], dst_v)
        plsc.subcore_barrier()

        def body(j, carry):
            pltpu.sync_copy(tbl_sh.at[src_v.at[j]], rows_v)
            pltpu.sync_copy(rows_v, acc_sh.at[dst_v.at[j]], add=True)
            return carry

        lax.fori_loop(0, CHUNKS, body, 0)
        plsc.subcore_barrier()
        pltpu.sync_copy(acc_sh.at[_acc_slice(s)],
                        out_hbm.at[c].at[_acc_slice(s)])

    return agg


_agg64 = _make_edge_agg(D_HID)
_agg1 = _make_edge_agg(1)


def _mm1_body(x_ref, w1_ref, degp_ref, t1_ref, dinv_ref):
    deg = degp_ref[0, :N, :] + degp_ref[1, :N, :] + 1.0
    dinv = lax.rsqrt(deg)
    mm = jnp.dot(x_ref[...], w1_ref[...], preferred_element_type=jnp.float32)
    t1_ref[:N, :] = mm * dinv
    t1_ref[N:, :] = jnp.zeros((N_ACC - N, D_HID), jnp.float32)
    dinv_ref[...] = dinv


def _mm2_body(aggp_ref, t1_ref, dinv_ref, w2_ref, b1_ref, t2_ref):
    aggsum = aggp_ref[0, :N, :] + aggp_ref[1, :N, :] + t1_ref[:N, :]
    h = jnp.maximum(aggsum * dinv_ref[...] + b1_ref[...], 0.0)
    mm = jnp.dot(h, w2_ref[...], preferred_element_type=jnp.float32)
    t2_ref[:N, :] = mm * dinv_ref[...]
    t2_ref[N:, :] = jnp.zeros((N_ACC - N, 1), jnp.float32)


def _fin_body(aggp_ref, t2_ref, dinv_ref, b2_ref, out_ref):
    a = aggp_ref[0, :N, :] + aggp_ref[1, :N, :] + t2_ref[:N, :]
    out_ref[...] = jax.nn.sigmoid(a * dinv_ref[...] + b2_ref[...])


def kernel(x, edge_index, W1, b1, W2, b2):
    src = edge_index[0].astype(jnp.int32)
    dst = edge_index[1].astype(jnp.int32)

    pad = E_PAD - E
    pr = jnp.arange(pad, dtype=jnp.int32)
    pad_src = (pr * 997) % N
    pad_dst = N + pr % (N_ACC - N)
    srcs = jnp.concatenate([src, pad_src]).reshape(NW, CHUNKS, K)
    dsts = jnp.concatenate([dst, pad_dst]).reshape(NW, CHUNKS, K)

    zeros64 = jnp.zeros((N_ACC, D_HID), jnp.float32)
    zeros1 = jnp.zeros((N_ACC, 1), jnp.float32)
    ones_tbl = jnp.ones((N_ACC, 1), jnp.float32)

    degp = _agg1(ones_tbl, srcs, dsts, zeros1)

    t1, dinv = pl.pallas_call(
        _mm1_body,
        out_shape=(
            jax.ShapeDtypeStruct((N_ACC, D_HID), jnp.float32),
            jax.ShapeDtypeStruct((N, 1), jnp.float32),
        ),
    )(x, W1, degp)

    agg1p = _agg64(t1, srcs, dsts, zeros64)

    t2 = pl.pallas_call(
        _mm2_body,
        out_shape=jax.ShapeDtypeStruct((N_ACC, 1), jnp.float32),
    )(agg1p, t1, dinv, W2, b1)

    agg2p = _agg1(t2, srcs, dsts, zeros1)

    out = pl.pallas_call(
        _fin_body,
        out_shape=jax.ShapeDtypeStruct((N, D_OUT), jnp.float32),
    )(agg2p, t2, dinv, b2)
    return out
